# Initial kernel scaffold; baseline (speedup 1.0000x reference)
import jax
import jax.numpy as jnp
from jax.experimental import pallas as pl


def kernel(input, mean, std, pconv_weight):
    raise NotImplementedError("write your pallas kernel here")


# speedup vs baseline: 4.4893x; 4.4893x over previous
import functools

import jax
import jax.numpy as jnp
from jax import lax
from jax.experimental import pallas as pl
from jax.experimental.pallas import tpu as pltpu
from jax.experimental.pallas import tpu_sc as plsc

H = 224
W = 224
L = H * W
C = 96
OC = 384
KNN = 5
NSEL = 9
PAD = KNN // 2
ZERO_ROW = L


def _shift_plane(a, dy, dx):
    b = a
    if dy > 0:
        b = jnp.concatenate([b[dy:, :], jnp.zeros((dy, W), b.dtype)], axis=0)
    elif dy < 0:
        b = jnp.concatenate([jnp.zeros((-dy, W), b.dtype), b[:H + dy, :]],
                            axis=0)
    if dx > 0:
        b = jnp.concatenate([b[:, dx:], jnp.zeros((H, dx), b.dtype)], axis=1)
    elif dx < 0:
        b = jnp.concatenate([jnp.zeros((H, -dx), b.dtype), b[:, :W + dx]],
                            axis=1)
    return b


def _topk_kernel(ch0_ref, ms_ref, idx_ref):
    mean = ms_ref[0, 0]
    std = ms_ref[0, 1]
    norm = (ch0_ref[...] * std + mean).astype(jnp.bfloat16).astype(
        jnp.float32)

    ys = lax.broadcasted_iota(jnp.int32, (H, W), 0)
    xs = lax.broadcasted_iota(jnp.int32, (H, W), 1)

    diffs = []
    lins = []
    for ky in range(KNN):
        for kx in range(KNN):
            dy, dx = ky - PAD, kx - PAD
            v = _shift_plane(norm, dy, dx)
            d = jnp.abs(v - norm)
            if dy == 0 and dx == 0:
                d = jnp.full((H, W), -1.0, dtype=jnp.float32)
            diffs.append(d)
            yy = ys + dy
            xx = xs + dx
            ok = (yy >= 0) & (yy < H) & (xx >= 0) & (xx < W)
            lins.append(jnp.where(ok, yy * W + xx, ZERO_ROW))

    big = jnp.float32(jnp.inf)
    for j in range(NSEL):
        m = functools.reduce(jnp.minimum, diffs)
        found = jnp.zeros((H, W), dtype=jnp.bool_)
        idx_j = jnp.full((H, W), ZERO_ROW, dtype=jnp.int32)
        for i in range(KNN * KNN):
            take = jnp.logical_and(jnp.logical_not(found), diffs[i] == m)
            found = jnp.logical_or(found, take)
            idx_j = jnp.where(take, lins[i], idx_j)
            diffs[i] = jnp.where(take, big, diffs[i])
        idx_ref[j, :, :] = idx_j


def _topk_indices(ch0, mean, std):
    ms = jnp.stack([mean[0], std[0]]).reshape(1, 2)
    return pl.pallas_call(
        _topk_kernel,
        out_shape=jax.ShapeDtypeStruct((NSEL, H, W), jnp.int32),
        in_specs=[
            pl.BlockSpec(memory_space=pltpu.VMEM),
            pl.BlockSpec(memory_space=pltpu.SMEM),
        ],
        out_specs=pl.BlockSpec(memory_space=pltpu.VMEM),
    )(ch0, ms)


NROWS = L * NSEL
NCORES = 2
NSUB = 16
NWORKERS = NCORES * NSUB
ROWS_PER_W = NROWS // NWORKERS
CHUNK = 504
NCHUNK = ROWS_PER_W // CHUNK


def _gather_body(table_hbm, idx_hbm, out_hbm, idx_v, rows_v, sem):
    wid = lax.axis_index("s") * NCORES + lax.axis_index("c")
    base = wid * ROWS_PER_W

    def step(k, carry):
        off = base + k * CHUNK
        pltpu.sync_copy(idx_hbm.at[pl.ds(off, CHUNK)], idx_v)
        pltpu.async_copy(table_hbm.at[idx_v], rows_v, sem).wait()
        pltpu.sync_copy(rows_v, out_hbm.at[pl.ds(off, CHUNK)])
        return carry

    lax.fori_loop(0, NCHUNK, step, 0)


@functools.cache
def _make_sc_gather():
    return pl.kernel(
        _gather_body,
        out_type=jax.ShapeDtypeStruct((NROWS, C), jnp.float32),
        mesh=plsc.VectorSubcoreMesh(core_axis_name="c", subcore_axis_name="s"),
        scratch_types=[
            pltpu.VMEM((CHUNK,), jnp.int32),
            pltpu.VMEM((CHUNK, C), jnp.float32),
            pltpu.SemaphoreType.DMA,
        ],
        compiler_params=pltpu.CompilerParams(use_tc_tiling_on_sc=False),
    )


LT = 512
NT = L // LT


def _matmul_kernel(w_ref, g_ref, out_ref):
    g = g_ref[...].astype(jnp.bfloat16)
    out_ref[...] = lax.dot_general(
        w_ref[...], g, (((1,), (1,)), ((), ())),
        preferred_element_type=jnp.float32)


def _matmul(w2_bf16, g_flat):
    return pl.pallas_call(
        _matmul_kernel,
        grid=(NT,),
        in_specs=[
            pl.BlockSpec((OC, NSEL * C), lambda i: (0, 0)),
            pl.BlockSpec((LT, NSEL * C), lambda i: (i, 0)),
        ],
        out_specs=pl.BlockSpec((OC, LT), lambda i: (0, i)),
        out_shape=jax.ShapeDtypeStruct((OC, L), jnp.float32),
    )(w2_bf16, g_flat)


def kernel(input, mean, std, pconv_weight):
    x = input.reshape(C, L)
    table = jnp.concatenate(
        [x.T, jnp.zeros((1, C), dtype=jnp.float32)], axis=0)

    idx9 = _topk_indices(input[0, 0], mean, std)
    flat_idx = idx9.reshape(NSEL, L).T.reshape(NROWS)

    g = _make_sc_gather()(table, flat_idx)
    g_flat = g.reshape(L, NSEL * C)

    w_flat = pconv_weight.reshape(OC, C * NSEL)
    w2 = w_flat.reshape(OC, C, NSEL).transpose(0, 2, 1).reshape(OC, NSEL * C)
    out2d = _matmul(w2.astype(jnp.bfloat16), g_flat)
    return out2d.reshape(1, OC, H, W)
